# baseline (speedup 1.0000x reference)
import functools

import jax
import jax.numpy as jnp
from jax import lax
from jax.experimental import pallas as pl
from jax.experimental.pallas import tpu as pltpu
from jax.experimental.pallas import tpu_sc as plsc

HIDDEN = 64
BATCH = 16384
MARGIN = 1.0
LANES = 16
NGRP = HIDDEN // LANES

NC = 2
NS = 16
NW = NC * NS

ROWS = 2 * BATCH
RPW = ROWS // NW
CHUNK = 128
NCHUNK = RPW // CHUNK


def _sc_body(ent, rel, hidx, tidx, ridx, out,
             idx_h, idx_t, idx_r, hv, tv, rv, acc, sem):
    wid = lax.axis_index("s") * NC + lax.axis_index("c")
    base = wid * RPW
    for c in range(NCHUNK):
        off = c * CHUNK
        pltpu.sync_copy(hidx.at[pl.ds(base + off, CHUNK)], idx_h)
        pltpu.sync_copy(tidx.at[pl.ds(base + off, CHUNK)], idx_t)
        pltpu.sync_copy(ridx.at[pl.ds(base + off, CHUNK)], idx_r)
        ch = pltpu.async_copy(ent.at[idx_h], hv, sem)
        ct = pltpu.async_copy(ent.at[idx_t], tv, sem)
        cr = pltpu.async_copy(rel.at[idx_r], rv, sem)
        ch.wait()
        ct.wait()
        cr.wait()

        def body(i, carry, off=off):
            a = None
            for k in range(NGRP):
                sl = pl.ds(k * LANES, LANES)
                d = hv[i, sl] + rv[i, sl] - tv[i, sl]
                sq = d * d
                a = sq if a is None else a + sq
            acc[off + i, :] = a
            return carry

        lax.fori_loop(0, CHUNK, body, 0)
    pltpu.sync_copy(acc, out.at[pl.ds(base, RPW)])


_sc_partials = functools.partial(
    pl.kernel,
    out_type=jax.ShapeDtypeStruct((ROWS, LANES), jnp.float32),
    mesh=plsc.VectorSubcoreMesh(core_axis_name="c", subcore_axis_name="s"),
    compiler_params=pltpu.CompilerParams(use_tc_tiling_on_sc=False),
    scratch_types=[
        pltpu.VMEM((CHUNK,), jnp.int32),
        pltpu.VMEM((CHUNK,), jnp.int32),
        pltpu.VMEM((CHUNK,), jnp.int32),
        pltpu.VMEM((CHUNK, HIDDEN), jnp.float32),
        pltpu.VMEM((CHUNK, HIDDEN), jnp.float32),
        pltpu.VMEM((CHUNK, HIDDEN), jnp.float32),
        pltpu.VMEM((RPW, LANES), jnp.float32),
        pltpu.SemaphoreType.DMA,
    ],
)(_sc_body)


def _tc_finish(parts_ref, out_ref):
    x = parts_ref[...]
    s = jnp.sum(x, axis=-1)
    sc = jnp.sqrt(s)
    val = jnp.maximum(sc[0] - sc[1] + MARGIN, 0.0)
    out_ref[0, 0] = jnp.sum(val)


def kernel(pos_h, pos_r, pos_t, neg_h, neg_r, neg_t,
           entity_embeddings, relation_embeddings):
    hidx = jnp.concatenate([pos_h, neg_h]).astype(jnp.int32)
    tidx = jnp.concatenate([pos_t, neg_t]).astype(jnp.int32)
    ridx = jnp.concatenate([pos_r[:, 0], neg_r[:, 0]]).astype(jnp.int32)
    parts = _sc_partials(entity_embeddings, relation_embeddings,
                         hidx, tidx, ridx)
    loss = pl.pallas_call(
        _tc_finish,
        out_shape=jax.ShapeDtypeStruct((1, 1), jnp.float32),
        out_specs=pl.BlockSpec(memory_space=pltpu.SMEM),
    )(parts.reshape(2, BATCH, LANES))
    return loss.reshape(())


# speedup vs baseline: 2.0569x; 2.0569x over previous
import functools

import jax
import jax.numpy as jnp
from jax import lax
from jax.experimental import pallas as pl
from jax.experimental.pallas import tpu as pltpu
from jax.experimental.pallas import tpu_sc as plsc

ENTITY_N = 1000000
RELATION_N = 1000
HIDDEN = 64
BATCH = 16384
MARGIN = 1.0
LANES = 16
NGRP = HIDDEN // LANES
SUB = 8

NC = 2
NS = 16
NW = NC * NS

ROWS = 2 * BATCH
RPW = ROWS // NW
CHUNK = 64
NCHUNK = RPW // CHUNK
OUT_W = 128
OUT_RPW = RPW * LANES // OUT_W
OUT_CPC = CHUNK * LANES // OUT_W

ENT_T = ENTITY_N // SUB
REL_T = RELATION_N // SUB


def _sc_body(ent3, rel3, hidx, tidx, ridx, out,
             hrows, trows, rrows, ih, it, ir, acc, sem):
    wid = lax.axis_index("s") * NC + lax.axis_index("c")
    base = wid * RPW
    for c in range(NCHUNK):
        off = c * CHUNK
        pltpu.sync_copy(hidx.at[pl.ds(base + off, CHUNK)], ih.at[pl.ds(0, CHUNK)])
        pltpu.sync_copy(tidx.at[pl.ds(base + off, CHUNK)], it.at[pl.ds(0, CHUNK)])
        pltpu.sync_copy(ridx.at[pl.ds(base + off, CHUNK)], ir.at[pl.ds(0, CHUNK)])

        def fire(j, carry):
            vh = ih[pl.ds(j, LANES)][0]
            pltpu.async_copy(ent3.at[vh >> 3, vh & 7], hrows.at[j], sem)
            vt = it[pl.ds(j, LANES)][0]
            pltpu.async_copy(ent3.at[vt >> 3, vt & 7], trows.at[j], sem)
            vr = ir[pl.ds(j, LANES)][0]
            pltpu.async_copy(rel3.at[vr >> 3, vr & 7], rrows.at[j], sem)
            return carry

        lax.fori_loop(0, CHUNK, fire, 0)

        def drain(j, carry):
            pltpu.make_async_copy(ent3.at[0, 0], hrows.at[0], sem).wait()
            pltpu.make_async_copy(ent3.at[0, 0], trows.at[0], sem).wait()
            pltpu.make_async_copy(ent3.at[0, 0], rrows.at[0], sem).wait()
            return carry

        lax.fori_loop(0, CHUNK, drain, 0)

        def cbody(j, carry, off=off):
            a = None
            for k in range(NGRP):
                sl = pl.ds(k * LANES, LANES)
                d = hrows[j, sl] - trows[j, sl] + rrows[j, sl]
                sq = d * d
                a = sq if a is None else a + sq
            acc[j >> 3, pl.ds((j & 7) * LANES, LANES)] = a
            return carry

        lax.fori_loop(0, CHUNK, cbody, 0)
        pltpu.sync_copy(
            acc, out.at[pl.ds(wid * OUT_RPW + c * OUT_CPC, OUT_CPC)])


_sc_partials = functools.partial(
    pl.kernel,
    out_type=jax.ShapeDtypeStruct((ROWS * LANES // OUT_W, OUT_W), jnp.float32),
    mesh=plsc.VectorSubcoreMesh(core_axis_name="c", subcore_axis_name="s"),
    scratch_types=[
        pltpu.VMEM((CHUNK, HIDDEN), jnp.float32),
        pltpu.VMEM((CHUNK, HIDDEN), jnp.float32),
        pltpu.VMEM((CHUNK, HIDDEN), jnp.float32),
        pltpu.VMEM((CHUNK + LANES,), jnp.int32),
        pltpu.VMEM((CHUNK + LANES,), jnp.int32),
        pltpu.VMEM((CHUNK + LANES,), jnp.int32),
        pltpu.VMEM((OUT_CPC, OUT_W), jnp.float32),
        pltpu.SemaphoreType.DMA,
    ],
)(_sc_body)


def _tc_finish(parts_ref, out_ref):
    x = parts_ref[...]
    g = lax.broadcasted_iota(jnp.int32, (OUT_W, OUT_W // LANES), 0)
    h = lax.broadcasted_iota(jnp.int32, (OUT_W, OUT_W // LANES), 1)
    m = (g // LANES == h).astype(jnp.float32)
    s = jax.lax.dot_general(x, m, (((1,), (0,)), ((), ())),
                            preferred_element_type=jnp.float32)
    sc = jnp.sqrt(s)
    half = sc.shape[0] // 2
    val = jnp.maximum(sc[:half] - sc[half:] + MARGIN, 0.0)
    out_ref[0, 0] = jnp.sum(val)


def kernel(pos_h, pos_r, pos_t, neg_h, neg_r, neg_t,
           entity_embeddings, relation_embeddings):
    hidx = jnp.concatenate([pos_h, neg_h]).astype(jnp.int32)
    tidx = jnp.concatenate([pos_t, neg_t]).astype(jnp.int32)
    ridx = jnp.concatenate([pos_r[:, 0], neg_r[:, 0]]).astype(jnp.int32)
    ent3 = entity_embeddings.reshape(ENT_T, SUB, HIDDEN)
    rel3 = relation_embeddings.reshape(REL_T, SUB, HIDDEN)
    parts = _sc_partials(ent3, rel3, hidx, tidx, ridx)
    loss = pl.pallas_call(
        _tc_finish,
        out_shape=jax.ShapeDtypeStruct((1, 1), jnp.float32),
        out_specs=pl.BlockSpec(memory_space=pltpu.SMEM),
    )(parts)
    return loss.reshape(())
